# Initial kernel scaffold; baseline (speedup 1.0000x reference)
import jax
import jax.numpy as jnp
from jax.experimental import pallas as pl


def kernel(x, pos_table):
    raise NotImplementedError("write your pallas kernel here")


# speedup vs baseline: 1.6885x; 1.6885x over previous
import jax
import jax.numpy as jnp
from jax.experimental import pallas as pl


_BLK_L = 1024


def _add_kernel(x_ref, pos_ref, o_ref):
    o_ref[...] = x_ref[...] + pos_ref[...][None]


def kernel(x, pos_table):
    B, L, D = x.shape
    blk_l = _BLK_L if L % _BLK_L == 0 else L
    grid = (L // blk_l, B)
    return pl.pallas_call(
        _add_kernel,
        grid=grid,
        in_specs=[
            pl.BlockSpec((1, blk_l, D), lambda l, b: (b, l, 0)),
            pl.BlockSpec((blk_l, D), lambda l, b: (l, 0)),
        ],
        out_specs=pl.BlockSpec((1, blk_l, D), lambda l, b: (b, l, 0)),
        out_shape=jax.ShapeDtypeStruct((B, L, D), x.dtype),
    )(x, pos_table[:L])


# speedup vs baseline: 1.7955x; 1.0634x over previous
import jax
import jax.numpy as jnp
from jax.experimental import pallas as pl


_BLK_L = 2048


def _add_kernel(x_ref, pos_ref, o_ref):
    o_ref[...] = x_ref[...] + pos_ref[...][None]


def kernel(x, pos_table):
    B, L, D = x.shape
    blk_l = _BLK_L if L % _BLK_L == 0 else L
    grid = (L // blk_l, B)
    return pl.pallas_call(
        _add_kernel,
        grid=grid,
        in_specs=[
            pl.BlockSpec((1, blk_l, D), lambda l, b: (b, l, 0)),
            pl.BlockSpec((blk_l, D), lambda l, b: (l, 0)),
        ],
        out_specs=pl.BlockSpec((1, blk_l, D), lambda l, b: (b, l, 0)),
        out_shape=jax.ShapeDtypeStruct((B, L, D), x.dtype),
    )(x, pos_table[:L])
